# Initial kernel scaffold; baseline (speedup 1.0000x reference)
import jax
import jax.numpy as jnp
from jax.experimental import pallas as pl


def kernel(x, edge_index, batch, W1, b1, W2, b2, Wfc, bfc, Wr1, br1, Wr2, br2, fusion_weight):
    raise NotImplementedError("write your pallas kernel here")


# speedup vs baseline: 41.1091x; 41.1091x over previous
import functools

import jax
import jax.numpy as jnp
from jax import lax
from jax.experimental import pallas as pl
from jax.experimental.pallas import tpu as pltpu
from jax.experimental.pallas import tpu_sc as plsc

N = 50176
E = 802816
G = 64

NC = 2
NS = 16
SUB = 128
UN = 4
CH = SUB * UN
NPT = N // NS

_EPT_A = E // (NC * NS)
_ITER_A = _EPT_A // CH
_EPT_C = E // NS
_ITER_C = _EPT_C // CH

_mesh = plsc.VectorSubcoreMesh(core_axis_name="c", subcore_axis_name="s")


@functools.partial(
    pl.kernel,
    out_type=jax.ShapeDtypeStruct((NC * N,), jnp.float32),
    mesh=_mesh,
    scratch_types=[
        pltpu.VMEM((UN, SUB), jnp.int32),
        pltpu.VMEM((SUB,), jnp.float32),
        pltpu.VMEM((NPT,), jnp.float32),
        pltpu.VMEM_SHARED((N,), jnp.float32),
        pltpu.SemaphoreType.DMA,
    ],
)
def _sc_deg(dm_hbm, ones_hbm, zeros_hbm, out_hbm, didx, ones_v, stage, acc, sem):
    c = lax.axis_index("c")
    s = lax.axis_index("s")
    wid = s * NC + c
    pltpu.sync_copy(zeros_hbm, stage)
    pltpu.sync_copy(stage, acc.at[pl.ds(s * NPT, NPT)])
    pltpu.sync_copy(ones_hbm, ones_v)
    plsc.subcore_barrier()
    row0 = wid * (_EPT_A // SUB)

    def body(k, carry):
        pltpu.sync_copy(dm_hbm.at[pl.ds(row0 + k * UN, UN)], didx)
        descs = [
            pltpu.async_copy(ones_v, acc.at[didx.at[j]], sem, add=True)
            for j in range(UN)
        ]
        for dsc in descs:
            dsc.wait()
        return carry

    lax.fori_loop(0, _ITER_A, body, 0)
    plsc.subcore_barrier()
    pltpu.sync_copy(acc.at[pl.ds(s * NPT, NPT)], stage)
    pltpu.sync_copy(stage, out_hbm.at[pl.ds(c * N + s * NPT, NPT)])


@functools.partial(
    pl.kernel,
    out_type=jax.ShapeDtypeStruct((NC * N,), jnp.float32),
    mesh=_mesh,
    scratch_types=[
        pltpu.VMEM((UN, SUB), jnp.int32),
        pltpu.VMEM((UN, SUB), jnp.int32),
        pltpu.VMEM((UN, SUB), jnp.float32),
        pltpu.VMEM((NPT,), jnp.float32),
        pltpu.VMEM_SHARED((N,), jnp.float32),
        pltpu.SemaphoreType.DMA,
        pltpu.SemaphoreType.DMA,
    ],
)
def _sc_scat1(sm_hbm, dm_hbm, p_hbm, zeros_hbm, out_hbm,
              sidx, didx, val, stage, acc, gsem, ssem):
    c = lax.axis_index("c")
    s = lax.axis_index("s")
    wid = s * NC + c
    pltpu.sync_copy(zeros_hbm, stage)
    pltpu.sync_copy(stage, acc.at[pl.ds(s * NPT, NPT)])
    plsc.subcore_barrier()
    row0 = wid * (_EPT_A // SUB)

    def body(k, carry):
        pltpu.sync_copy(sm_hbm.at[pl.ds(row0 + k * UN, UN)], sidx)
        pltpu.sync_copy(dm_hbm.at[pl.ds(row0 + k * UN, UN)], didx)
        gd = [pltpu.async_copy(p_hbm.at[sidx.at[j]], val.at[j], gsem)
              for j in range(UN)]
        for dsc in gd:
            dsc.wait()
        sd = [pltpu.async_copy(val.at[j], acc.at[didx.at[j]], ssem, add=True)
              for j in range(UN)]
        for dsc in sd:
            dsc.wait()
        return carry

    lax.fori_loop(0, _ITER_A, body, 0)
    plsc.subcore_barrier()
    pltpu.sync_copy(acc.at[pl.ds(s * NPT, NPT)], stage)
    pltpu.sync_copy(stage, out_hbm.at[pl.ds(c * N + s * NPT, NPT)])


@functools.partial(
    pl.kernel,
    out_type=jax.ShapeDtypeStruct((NC * N, 32), jnp.float32),
    mesh=_mesh,
    compiler_params=pltpu.CompilerParams(use_tc_tiling_on_sc=False),
    scratch_types=[
        pltpu.VMEM((UN, SUB), jnp.int32),
        pltpu.VMEM((UN, SUB), jnp.int32),
        pltpu.VMEM((UN, SUB, 32), jnp.float32),
        pltpu.VMEM((NPT // 8, 32), jnp.float32),
        pltpu.VMEM_SHARED((N, 32), jnp.float32),
        pltpu.SemaphoreType.DMA,
        pltpu.SemaphoreType.DMA,
    ],
)
def _sc_scat2(s2m_hbm, dm_hbm, q_hbm, zeros_hbm, out_hbm,
              sidx, didx, rows, stage, acc, gsem, ssem):
    c = lax.axis_index("c")
    s = lax.axis_index("s")
    pltpu.sync_copy(zeros_hbm, stage)
    for j in range(8):
        pltpu.sync_copy(stage, acc.at[pl.ds(s * NPT + j * (NPT // 8), NPT // 8)])
    plsc.subcore_barrier()
    row0 = c * (E // SUB) + s * (_EPT_C // SUB)
    drow0 = s * (_EPT_C // SUB)

    def body(k, carry):
        pltpu.sync_copy(s2m_hbm.at[pl.ds(row0 + k * UN, UN)], sidx)
        pltpu.sync_copy(dm_hbm.at[pl.ds(drow0 + k * UN, UN)], didx)
        gd = [pltpu.async_copy(q_hbm.at[sidx.at[j]], rows.at[j], gsem)
              for j in range(UN)]
        for dsc in gd:
            dsc.wait()
        sd = [pltpu.async_copy(rows.at[j], acc.at[didx.at[j]], ssem, add=True)
              for j in range(UN)]
        for dsc in sd:
            dsc.wait()
        return carry

    lax.fori_loop(0, _ITER_C, body, 0)
    plsc.subcore_barrier()
    for j in range(8):
        pltpu.sync_copy(acc.at[pl.ds(s * NPT + j * (NPT // 8), NPT // 8)], stage)
        pltpu.sync_copy(stage,
                        out_hbm.at[pl.ds(c * N + s * NPT + j * (NPT // 8),
                                         NPT // 8)])


_R = N // 128


def _tc_prep_body(degp_ref, x_ref, dis_ref, p_ref):
    deg = degp_ref[0] + degp_ref[1] + 1.0
    dis = lax.rsqrt(deg)
    dis_ref[...] = dis
    p_ref[...] = dis * x_ref[...]


_tc_prep = pl.pallas_call(
    _tc_prep_body,
    out_shape=(jax.ShapeDtypeStruct((_R, 128), jnp.float32),
               jax.ShapeDtypeStruct((_R, 128), jnp.float32)),
)

_BN = 3136


def _tc_h1_body(scp_ref, p_ref, dis_ref, w1a_ref, w1b_ref, b1a_ref, b1b_ref,
                q_ref):
    dis = dis_ref[...]
    agg1 = (scp_ref[0] + scp_ref[1] + p_ref[...]) * dis
    q_ref[0] = dis * jnp.tanh(agg1 * w1a_ref[...] + b1a_ref[...])
    q_ref[1] = dis * jnp.tanh(agg1 * w1b_ref[...] + b1b_ref[...])


_tc_h1 = pl.pallas_call(
    _tc_h1_body,
    grid=(N // _BN,),
    in_specs=[
        pl.BlockSpec((2, _BN, 1), lambda i: (0, i, 0)),
        pl.BlockSpec((_BN, 1), lambda i: (i, 0)),
        pl.BlockSpec((_BN, 1), lambda i: (i, 0)),
        pl.BlockSpec((1, 32), lambda i: (0, 0)),
        pl.BlockSpec((1, 32), lambda i: (0, 0)),
        pl.BlockSpec((1, 32), lambda i: (0, 0)),
        pl.BlockSpec((1, 32), lambda i: (0, 0)),
    ],
    out_specs=pl.BlockSpec((2, _BN, 32), lambda i: (0, i, 0)),
    out_shape=jax.ShapeDtypeStruct((2, N, 32), jnp.float32),
)


def _tc_final_body(sc2_ref, q_ref, dis_ref, batch_ref, w2a_ref, w2b_ref,
                   b2_ref, xr_ref, wfc_ref, bfc_ref, wr1_ref, br1_ref,
                   wr2_ref, br2_ref, fw_ref, out_ref, sums_ref, cnt_ref):
    i = pl.program_id(0)

    @pl.when(i == 0)
    def _init():
        sums_ref[...] = jnp.zeros_like(sums_ref)
        cnt_ref[...] = jnp.zeros_like(cnt_ref)

    dis = dis_ref[...]
    a0 = (sc2_ref[0] + q_ref[0]) * dis
    a1 = (sc2_ref[1] + q_ref[1]) * dis
    h2 = jnp.tanh(
        jnp.dot(a0, w2a_ref[...], preferred_element_type=jnp.float32)
        + jnp.dot(a1, w2b_ref[...], preferred_element_type=jnp.float32)
        + b2_ref[...])
    oh = (batch_ref[0] ==
          lax.broadcasted_iota(jnp.int32, (G, _BN), 0)).astype(jnp.float32)
    sums_ref[...] += jnp.dot(oh, h2, preferred_element_type=jnp.float32)
    cnt_ref[...] += jnp.sum(oh, axis=1, keepdims=True)

    @pl.when(i == N // _BN - 1)
    def _fin():
        pool = sums_ref[...] / jnp.maximum(cnt_ref[...], 1.0)
        h = jnp.dot(pool, wfc_ref[...],
                    preferred_element_type=jnp.float32) + bfc_ref[...]
        r1 = jnp.maximum(
            jnp.dot(xr_ref[...], wr1_ref[...],
                    preferred_element_type=jnp.float32) + br1_ref[...], 0.0)
        r = jnp.dot(r1, wr2_ref[...],
                    preferred_element_type=jnp.float32) + br2_ref[...]
        fw = jax.nn.sigmoid(fw_ref[0, 0])
        fused = fw * h + (1.0 - fw) * r
        m = jnp.max(fused, axis=1, keepdims=True)
        lse = m + jnp.log(jnp.sum(jnp.exp(fused - m), axis=1, keepdims=True))
        out_ref[...] = fused - lse


_tc_final = pl.pallas_call(
    _tc_final_body,
    grid=(N // _BN,),
    in_specs=[
        pl.BlockSpec((2, _BN, 32), lambda i: (0, i, 0)),
        pl.BlockSpec((2, _BN, 32), lambda i: (0, i, 0)),
        pl.BlockSpec((_BN, 1), lambda i: (i, 0)),
        pl.BlockSpec((1, 1, _BN), lambda i: (i, 0, 0)),
        pl.BlockSpec((32, 128), lambda i: (0, 0)),
        pl.BlockSpec((32, 128), lambda i: (0, 0)),
        pl.BlockSpec((1, 128), lambda i: (0, 0)),
        pl.BlockSpec((G, 784), lambda i: (0, 0)),
        pl.BlockSpec((128, 10), lambda i: (0, 0)),
        pl.BlockSpec((1, 10), lambda i: (0, 0)),
        pl.BlockSpec((784, 128), lambda i: (0, 0)),
        pl.BlockSpec((1, 128), lambda i: (0, 0)),
        pl.BlockSpec((128, 10), lambda i: (0, 0)),
        pl.BlockSpec((1, 10), lambda i: (0, 0)),
        pl.BlockSpec((1, 1), lambda i: (0, 0)),
    ],
    out_specs=pl.BlockSpec((G, 10), lambda i: (0, 0)),
    out_shape=jax.ShapeDtypeStruct((G, 10), jnp.float32),
    scratch_shapes=[
        pltpu.VMEM((G, 128), jnp.float32),
        pltpu.VMEM((G, 1), jnp.float32),
    ],
)


def kernel(x, edge_index, batch, W1, b1, W2, b2, Wfc, bfc, Wr1, br1, Wr2, br2,
           fusion_weight):
    src = edge_index[0]
    dst = edge_index[1]
    dm = dst.reshape(E // SUB, SUB)
    sm = src.reshape(E // SUB, SUB)
    s2m = jnp.concatenate([src, src + N]).reshape(NC * E // SUB, SUB)

    zeros_n = jnp.zeros((NPT,), jnp.float32)
    zeros_n32 = jnp.zeros((NPT // 8, 32), jnp.float32)
    ones_sub = jnp.ones((SUB,), jnp.float32)

    degp = _sc_deg(dm, ones_sub, zeros_n)
    dis2d, p2d = _tc_prep(degp.reshape(NC, _R, 128), x.reshape(_R, 128))

    scat1p = _sc_scat1(sm, dm, p2d.reshape(N), zeros_n)

    dis_col = dis2d.reshape(N, 1)
    q = _tc_h1(scat1p.reshape(NC, N, 1), p2d.reshape(N, 1), dis_col,
               W1[:, :32], W1[:, 32:], b1[:32].reshape(1, 32),
               b1[32:].reshape(1, 32))

    scat2 = _sc_scat2(s2m, dm, q.reshape(NC * N, 32), zeros_n32)

    out = _tc_final(scat2.reshape(NC, N, 32), q, dis_col,
                    batch.reshape(N // _BN, 1, _BN),
                    W2[:32], W2[32:], b2.reshape(1, 128), x.reshape(G, 784),
                    Wfc, bfc.reshape(1, 10), Wr1, br1.reshape(1, 128), Wr2,
                    br2.reshape(1, 10), fusion_weight.reshape(1, 1))
    return out


# speedup vs baseline: 64.0524x; 1.5581x over previous
import functools

import jax
import jax.numpy as jnp
from jax import lax
from jax.experimental import pallas as pl
from jax.experimental.pallas import tpu as pltpu
from jax.experimental.pallas import tpu_sc as plsc

N = 50176
E = 802816
G = 64

NC = 2
NS = 16
SUB = 128
UNA = 4
UNC = 4
NPT = N // NS
NCH = NPT // 32

_ROWS_T = E // (NS * SUB)
_ITER16 = _ROWS_T // UNA
_VPI = NPT // 16

_mesh = plsc.VectorSubcoreMesh(core_axis_name="c", subcore_axis_name="s")


@functools.partial(
    pl.kernel,
    out_type=(jax.ShapeDtypeStruct((NC * N, 32), jnp.float32),
              jax.ShapeDtypeStruct((NC * N,), jnp.float32)),
    mesh=_mesh,
    compiler_params=pltpu.CompilerParams(use_tc_tiling_on_sc=False,
                                         needs_layout_passes=False),
    scratch_types=[
        pltpu.VMEM((2, 2 * UNA, SUB), jnp.int32),
        pltpu.VMEM((2, UNA, SUB), jnp.int32),
        pltpu.VMEM((UNA, SUB), jnp.float32),
        pltpu.VMEM((SUB,), jnp.float32),
        pltpu.VMEM((NPT,), jnp.float32),
        pltpu.VMEM((NPT,), jnp.float32),
        pltpu.VMEM((NPT,), jnp.float32),
        pltpu.VMEM((64,), jnp.float32),
        pltpu.VMEM((64,), jnp.float32),
        pltpu.VMEM((2 * NCH, 32), jnp.float32),
        pltpu.VMEM_SHARED((N,), jnp.float32),
        pltpu.VMEM_SHARED((N,), jnp.float32),
        pltpu.SemaphoreType.DMA((2,)),
        pltpu.SemaphoreType.DMA((UNA,)),
        pltpu.SemaphoreType.DMA((UNA,)),
        pltpu.SemaphoreType.DMA((2,)),
    ],
)
def _sc_l1(sd_hbm, dm_hbm, x_hbm, zeros_hbm, ones_hbm, w1_hbm, b1_hbm,
           q_hbm, dis_hbm,
           idxb, didxb, val, ones_v, stage, xbuf, disbuf, wbuf, bbuf, qch,
           dacc, sacc, isem, gsem, ssem, qsem):
    c = lax.axis_index("c")
    s = lax.axis_index("s")
    pltpu.sync_copy(zeros_hbm, stage)
    pltpu.sync_copy(stage, dacc.at[pl.ds(s * NPT, NPT)])
    pltpu.sync_copy(stage, sacc.at[pl.ds(s * NPT, NPT)])
    pltpu.sync_copy(ones_hbm, ones_v)
    pltpu.sync_copy(w1_hbm, wbuf)
    pltpu.sync_copy(b1_hbm, bbuf)
    plsc.subcore_barrier()

    drow0 = s * _ROWS_T
    pltpu.async_copy(dm_hbm.at[pl.ds(drow0, UNA)], didxb.at[0], isem.at[0])

    def dbody(k, carry):
        b = lax.rem(k, 2)

        @pl.when(k + 1 < _ITER16)
        def _pref():
            pltpu.async_copy(dm_hbm.at[pl.ds(drow0 + (k + 1) * UNA, UNA)],
                             didxb.at[1 - b], isem.at[1 - b])

        pltpu.make_async_copy(dm_hbm.at[pl.ds(drow0 + k * UNA, UNA)],
                              didxb.at[b], isem.at[b]).wait()
        descs = [
            pltpu.async_copy(ones_v, dacc.at[didxb.at[b, j]], gsem.at[j],
                             add=True)
            for j in range(UNA)
        ]
        for dsc in descs:
            dsc.wait()
        return carry

    lax.fori_loop(0, _ITER16, dbody, 0)
    plsc.subcore_barrier()

    pltpu.sync_copy(dacc.at[pl.ds(s * NPT, NPT)], stage)
    pltpu.sync_copy(x_hbm.at[pl.ds(s * NPT, NPT)], xbuf)

    def vbody(k, carry):
        ix = pl.ds(k * 16, 16)
        d = stage[ix] + 1.0
        bits = plsc.bitcast(d, jnp.int32)
        y = plsc.bitcast(jnp.int32(0x5F3759DF) - (bits >> 1), jnp.float32)
        y = y * (1.5 - 0.5 * d * y * y)
        y = y * (1.5 - 0.5 * d * y * y)
        y = y * (1.5 - 0.5 * d * y * y)
        disbuf[ix] = y
        xbuf[ix] = y * xbuf[ix]
        return carry

    lax.fori_loop(0, _VPI, vbody, 0)
    pltpu.sync_copy(xbuf, dacc.at[pl.ds(s * NPT, NPT)])
    pltpu.sync_copy(disbuf, dis_hbm.at[pl.ds(c * N + s * NPT, NPT)])
    plsc.subcore_barrier()

    srow0 = s * 2 * _ROWS_T
    pltpu.async_copy(sd_hbm.at[pl.ds(srow0, 2 * UNA)], idxb.at[0], isem.at[0])

    def sbody(k, carry):
        b = lax.rem(k, 2)

        @pl.when(k + 1 < _ITER16)
        def _pref():
            pltpu.async_copy(
                sd_hbm.at[pl.ds(srow0 + (k + 1) * 2 * UNA, 2 * UNA)],
                idxb.at[1 - b], isem.at[1 - b])

        pltpu.make_async_copy(sd_hbm.at[pl.ds(srow0 + k * 2 * UNA, 2 * UNA)],
                              idxb.at[b], isem.at[b]).wait()
        gd = [pltpu.async_copy(dacc.at[idxb.at[b, 2 * j]], val.at[j],
                               gsem.at[j])
              for j in range(UNA)]
        sd = []
        for j in range(UNA):
            gd[j].wait()
            sd.append(pltpu.async_copy(val.at[j],
                                       sacc.at[idxb.at[b, 2 * j + 1]],
                                       ssem.at[j], add=True))
        for dsc in sd:
            dsc.wait()
        return carry

    lax.fori_loop(0, _ITER16, sbody, 0)
    plsc.subcore_barrier()

    pltpu.sync_copy(sacc.at[pl.ds(s * NPT, NPT)], stage)

    def abody(k, carry):
        ix = pl.ds(k * 16, 16)
        stage[ix] = disbuf[ix] * (stage[ix] + xbuf[ix])
        return carry

    lax.fori_loop(0, _VPI, abody, 0)

    wlo = wbuf[pl.ds(c * 32, 16)]
    whi = wbuf[pl.ds(c * 32 + 16, 16)]
    blo = bbuf[pl.ds(c * 32, 16)]
    bhi = bbuf[pl.ds(c * 32 + 16, 16)]

    def qbody(ch, carry):
        bb = lax.rem(ch, 2)

        @pl.when(ch >= 2)
        def _drain():
            pltpu.make_async_copy(
                qch.at[pl.ds(bb * NCH, NCH)],
                q_hbm.at[pl.ds(c * N + s * NPT + (ch - 2) * NCH, NCH)],
                qsem.at[bb]).wait()

        def nbody(i, carry2):
            node = jnp.full((16,), ch * NCH + i, jnp.int32)
            a = plsc.load_gather(stage, [node])
            dv = plsc.load_gather(disbuf, [node])
            r = bb * NCH + i
            for (w_, b_, g) in ((wlo, blo, 0), (whi, bhi, 1)):
                z = a * w_ + b_
                z = jnp.maximum(jnp.minimum(z, 15.0), -15.0)
                t = jnp.exp(z + z)
                qch[r, pl.ds(16 * g, 16)] = dv * (t - 1.0) / (t + 1.0)
            return carry2

        lax.fori_loop(0, NCH, nbody, 0)
        pltpu.async_copy(qch.at[pl.ds(bb * NCH, NCH)],
                         q_hbm.at[pl.ds(c * N + s * NPT + ch * NCH, NCH)],
                         qsem.at[bb])
        return carry

    lax.fori_loop(0, NPT // NCH, qbody, 0)
    for bb in range(2):
        pltpu.make_async_copy(
            qch.at[pl.ds(bb * NCH, NCH)],
            q_hbm.at[pl.ds(c * N + s * NPT + (30 + bb) * NCH, NCH)],
            qsem.at[bb]).wait()


@functools.partial(
    pl.kernel,
    out_type=jax.ShapeDtypeStruct((NC * N, 32), jnp.float32),
    mesh=_mesh,
    compiler_params=pltpu.CompilerParams(use_tc_tiling_on_sc=False,
                                         needs_layout_passes=False),
    scratch_types=[
        pltpu.VMEM((2, 2 * UNC, SUB), jnp.int32),
        pltpu.VMEM((UNC, SUB, 32), jnp.float32),
        pltpu.VMEM((NPT // 16, 32), jnp.float32),
        pltpu.VMEM((NPT,), jnp.float32),
        pltpu.VMEM_SHARED((N, 32), jnp.float32),
        pltpu.SemaphoreType.DMA((2,)),
        pltpu.SemaphoreType.DMA((UNC,)),
        pltpu.SemaphoreType.DMA((UNC,)),
    ],
)
def _sc_scat2(sd2_hbm, q_hbm, dis_hbm, out_hbm,
              idxb, rows, stage, disbuf, acc, isem, gsem, ssem):
    c = lax.axis_index("c")
    s = lax.axis_index("s")
    pltpu.sync_copy(dis_hbm.at[pl.ds(c * N + s * NPT, NPT)], disbuf)
    for j in range(16):
        pltpu.sync_copy(
            q_hbm.at[pl.ds(c * N + s * NPT + j * (NPT // 16), NPT // 16)],
            stage)
        pltpu.sync_copy(stage,
                        acc.at[pl.ds(s * NPT + j * (NPT // 16), NPT // 16)])
    plsc.subcore_barrier()
    row0 = c * 2 * (E // SUB) + s * 2 * _ROWS_T
    pltpu.async_copy(sd2_hbm.at[pl.ds(row0, 2 * UNC)], idxb.at[0], isem.at[0])

    def body(k, carry):
        b = lax.rem(k, 2)

        @pl.when(k + 1 < _ITER16)
        def _pref():
            pltpu.async_copy(
                sd2_hbm.at[pl.ds(row0 + (k + 1) * 2 * UNC, 2 * UNC)],
                idxb.at[1 - b], isem.at[1 - b])

        pltpu.make_async_copy(sd2_hbm.at[pl.ds(row0 + k * 2 * UNC, 2 * UNC)],
                              idxb.at[b], isem.at[b]).wait()
        gd = [pltpu.async_copy(q_hbm.at[idxb.at[b, 2 * j]], rows.at[j],
                               gsem.at[j])
              for j in range(UNC)]
        sd = []
        for j in range(UNC):
            gd[j].wait()
            sd.append(pltpu.async_copy(rows.at[j],
                                       acc.at[idxb.at[b, 2 * j + 1]],
                                       ssem.at[j], add=True))
        for dsc in sd:
            dsc.wait()
        return carry

    lax.fori_loop(0, _ITER16, body, 0)
    plsc.subcore_barrier()

    for j in range(16):
        pltpu.sync_copy(acc.at[pl.ds(s * NPT + j * (NPT // 16), NPT // 16)],
                        stage)

        def scbody(i, carry):
            dv = plsc.load_gather(
                disbuf, [jnp.full((16,), j * (NPT // 16) + i, jnp.int32)])
            stage[i, pl.ds(0, 16)] = dv * stage[i, pl.ds(0, 16)]
            stage[i, pl.ds(16, 16)] = dv * stage[i, pl.ds(16, 16)]
            return carry

        lax.fori_loop(0, NPT // 16, scbody, 0)
        pltpu.sync_copy(stage,
                        out_hbm.at[pl.ds(c * N + s * NPT + j * (NPT // 16),
                                         NPT // 16)])


_BP = 1568
_GRID_F = N // (4 * _BP)


def _tc_final_body(scp_ref, batch_ref, w2a_ref, w2b_ref, b2_ref, xr_ref,
                   wfc_ref, bfc_ref, wr1_ref, br1_ref, wr2_ref, br2_ref,
                   fw_ref, out_ref, sums_ref, cnt_ref):
    i = pl.program_id(0)

    @pl.when(i == 0)
    def _init():
        sums_ref[...] = jnp.zeros_like(sums_ref)
        cnt_ref[...] = jnp.zeros_like(cnt_ref)

    h2p = jnp.tanh(
        jnp.dot(scp_ref[0], w2a_ref[...], preferred_element_type=jnp.float32)
        + jnp.dot(scp_ref[1], w2b_ref[...],
                  preferred_element_type=jnp.float32)
        + b2_ref[...])
    bv = batch_ref[0]
    for l in range(4):
        ohl = (bv[l:l + 1, :] ==
               lax.broadcasted_iota(jnp.int32, (G, _BP), 0)
               ).astype(jnp.float32)
        sums_ref[...] += jnp.dot(ohl, h2p[:, 128 * l:128 * (l + 1)],
                                 preferred_element_type=jnp.float32)
        cnt_ref[...] += jnp.sum(ohl, axis=1, keepdims=True)

    @pl.when(i == _GRID_F - 1)
    def _fin():
        pool = sums_ref[...] / jnp.maximum(cnt_ref[...], 1.0)
        h = jnp.dot(pool, wfc_ref[...],
                    preferred_element_type=jnp.float32) + bfc_ref[...]
        r1 = jnp.maximum(
            jnp.dot(xr_ref[...], wr1_ref[...],
                    preferred_element_type=jnp.float32) + br1_ref[...], 0.0)
        r = jnp.dot(r1, wr2_ref[...],
                    preferred_element_type=jnp.float32) + br2_ref[...]
        fw = jax.nn.sigmoid(fw_ref[0, 0])
        fused = fw * h + (1.0 - fw) * r
        m = jnp.max(fused, axis=1, keepdims=True)
        lse = m + jnp.log(jnp.sum(jnp.exp(fused - m), axis=1, keepdims=True))
        out_ref[...] = fused - lse


_tc_final = pl.pallas_call(
    _tc_final_body,
    grid=(_GRID_F,),
    in_specs=[
        pl.BlockSpec((2, _BP, 128), lambda i: (0, i, 0)),
        pl.BlockSpec((1, 4, _BP), lambda i: (i, 0, 0)),
        pl.BlockSpec((128, 512), lambda i: (0, 0)),
        pl.BlockSpec((128, 512), lambda i: (0, 0)),
        pl.BlockSpec((1, 512), lambda i: (0, 0)),
        pl.BlockSpec((G, 784), lambda i: (0, 0)),
        pl.BlockSpec((128, 10), lambda i: (0, 0)),
        pl.BlockSpec((1, 10), lambda i: (0, 0)),
        pl.BlockSpec((784, 128), lambda i: (0, 0)),
        pl.BlockSpec((1, 128), lambda i: (0, 0)),
        pl.BlockSpec((128, 10), lambda i: (0, 0)),
        pl.BlockSpec((1, 10), lambda i: (0, 0)),
        pl.BlockSpec((1, 1), lambda i: (0, 0)),
    ],
    out_specs=pl.BlockSpec((G, 10), lambda i: (0, 0)),
    out_shape=jax.ShapeDtypeStruct((G, 10), jnp.float32),
    scratch_shapes=[
        pltpu.VMEM((G, 128), jnp.float32),
        pltpu.VMEM((G, 1), jnp.float32),
    ],
)


def kernel(x, edge_index, batch, W1, b1, W2, b2, Wfc, bfc, Wr1, br1, Wr2, br2,
           fusion_weight):
    from jax.scipy.linalg import block_diag

    src = edge_index[0]
    dst = edge_index[1]
    dm = dst.reshape(E // SUB, SUB)
    sm = src.reshape(E // SUB, SUB)
    sd1 = jnp.stack([sm, dm], axis=1).reshape(2 * (E // SUB), SUB)
    s2r = jnp.stack([sm, sm + N])
    dmr = jnp.broadcast_to(dm[None], (2,) + dm.shape)
    sd2 = jnp.stack([s2r, dmr], axis=2).reshape(4 * (E // SUB), SUB)

    zeros_n = jnp.zeros((NPT,), jnp.float32)
    ones_sub = jnp.ones((SUB,), jnp.float32)

    q_lin, dis2 = _sc_l1(sd1, dm, x.reshape(N), zeros_n, ones_sub,
                         W1.reshape(64), b1)
    agg2 = _sc_scat2(sd2, q_lin, dis2)

    w2a_t = block_diag(W2[:32], W2[:32], W2[:32], W2[:32])
    w2b_t = block_diag(W2[32:], W2[32:], W2[32:], W2[32:])
    b2rep = jnp.tile(b2, 4).reshape(1, 512)
    batch4 = batch.reshape(_GRID_F, _BP, 4).transpose(0, 2, 1)

    out = _tc_final(agg2.reshape(NC, N // 4, 128), batch4, w2a_t, w2b_t,
                    b2rep, x.reshape(G, 784), Wfc, bfc.reshape(1, 10), Wr1,
                    br1.reshape(1, 128), Wr2, br2.reshape(1, 10),
                    fusion_weight.reshape(1, 1))
    return out


# speedup vs baseline: 68.7073x; 1.0727x over previous
import functools

import jax
import jax.numpy as jnp
from jax import lax
from jax.experimental import pallas as pl
from jax.experimental.pallas import tpu as pltpu
from jax.experimental.pallas import tpu_sc as plsc

N = 50176
E = 802816
G = 64

NC = 2
NS = 16
SUB = 128
UNA = 8
UNC = 4
NPT = N // NS
NCH = NPT // 32

_ROWS_T = E // (NS * SUB)
_ITERA = _ROWS_T // UNA
_ITER16 = _ROWS_T // UNC
_VPI = NPT // 16

_mesh = plsc.VectorSubcoreMesh(core_axis_name="c", subcore_axis_name="s")


@functools.partial(
    pl.kernel,
    out_type=(jax.ShapeDtypeStruct((NC * N, 32), jnp.float32),
              jax.ShapeDtypeStruct((NC * N,), jnp.float32)),
    mesh=_mesh,
    compiler_params=pltpu.CompilerParams(use_tc_tiling_on_sc=False,
                                         needs_layout_passes=False),
    scratch_types=[
        pltpu.VMEM((2, 2 * UNA, SUB), jnp.int32),
        pltpu.VMEM((2, UNA, SUB), jnp.int32),
        pltpu.VMEM((UNA, SUB), jnp.float32),
        pltpu.VMEM((SUB,), jnp.float32),
        pltpu.VMEM((NPT,), jnp.float32),
        pltpu.VMEM((NPT,), jnp.float32),
        pltpu.VMEM((NPT,), jnp.float32),
        pltpu.VMEM((64,), jnp.float32),
        pltpu.VMEM((64,), jnp.float32),
        pltpu.VMEM((2 * NCH, 32), jnp.float32),
        pltpu.VMEM_SHARED((N,), jnp.float32),
        pltpu.VMEM_SHARED((N,), jnp.float32),
        pltpu.SemaphoreType.DMA((2,)),
        pltpu.SemaphoreType.DMA((UNA,)),
        pltpu.SemaphoreType.DMA((UNA,)),
        pltpu.SemaphoreType.DMA((2,)),
    ],
)
def _sc_l1(sd_hbm, dm_hbm, x_hbm, zeros_hbm, ones_hbm, w1_hbm, b1_hbm,
           q_hbm, dis_hbm,
           idxb, didxb, val, ones_v, stage, xbuf, disbuf, wbuf, bbuf, qch,
           dacc, sacc, isem, gsem, ssem, qsem):
    c = lax.axis_index("c")
    s = lax.axis_index("s")
    pltpu.sync_copy(zeros_hbm, stage)
    pltpu.sync_copy(stage, dacc.at[pl.ds(s * NPT, NPT)])
    pltpu.sync_copy(stage, sacc.at[pl.ds(s * NPT, NPT)])
    pltpu.sync_copy(ones_hbm, ones_v)
    pltpu.sync_copy(w1_hbm, wbuf)
    pltpu.sync_copy(b1_hbm, bbuf)
    plsc.subcore_barrier()

    drow0 = s * _ROWS_T
    pltpu.async_copy(dm_hbm.at[pl.ds(drow0, UNA)], didxb.at[0], isem.at[0])

    def dbody(k, carry):
        b = lax.rem(k, 2)

        @pl.when(k + 1 < _ITERA)
        def _pref():
            pltpu.async_copy(dm_hbm.at[pl.ds(drow0 + (k + 1) * UNA, UNA)],
                             didxb.at[1 - b], isem.at[1 - b])

        pltpu.make_async_copy(dm_hbm.at[pl.ds(drow0 + k * UNA, UNA)],
                              didxb.at[b], isem.at[b]).wait()
        descs = [
            pltpu.async_copy(ones_v, dacc.at[didxb.at[b, j]], gsem.at[j],
                             add=True)
            for j in range(UNA)
        ]
        for dsc in descs:
            dsc.wait()
        return carry

    lax.fori_loop(0, _ITERA, dbody, 0)
    plsc.subcore_barrier()

    pltpu.sync_copy(dacc.at[pl.ds(s * NPT, NPT)], stage)
    pltpu.sync_copy(x_hbm.at[pl.ds(s * NPT, NPT)], xbuf)

    def vbody(k, carry):
        ix = pl.ds(k * 16, 16)
        d = stage[ix] + 1.0
        bits = plsc.bitcast(d, jnp.int32)
        y = plsc.bitcast(jnp.int32(0x5F3759DF) - (bits >> 1), jnp.float32)
        y = y * (1.5 - 0.5 * d * y * y)
        y = y * (1.5 - 0.5 * d * y * y)
        y = y * (1.5 - 0.5 * d * y * y)
        disbuf[ix] = y
        xbuf[ix] = y * xbuf[ix]
        return carry

    lax.fori_loop(0, _VPI, vbody, 0)
    pltpu.sync_copy(xbuf, dacc.at[pl.ds(s * NPT, NPT)])
    pltpu.sync_copy(disbuf, dis_hbm.at[pl.ds(c * N + s * NPT, NPT)])
    plsc.subcore_barrier()

    srow0 = s * 2 * _ROWS_T
    pltpu.async_copy(sd_hbm.at[pl.ds(srow0, 2 * UNA)], idxb.at[0], isem.at[0])

    def sbody(k, carry):
        b = lax.rem(k, 2)

        @pl.when(k + 1 < _ITERA)
        def _pref():
            pltpu.async_copy(
                sd_hbm.at[pl.ds(srow0 + (k + 1) * 2 * UNA, 2 * UNA)],
                idxb.at[1 - b], isem.at[1 - b])

        pltpu.make_async_copy(sd_hbm.at[pl.ds(srow0 + k * 2 * UNA, 2 * UNA)],
                              idxb.at[b], isem.at[b]).wait()
        gd = [pltpu.async_copy(dacc.at[idxb.at[b, 2 * j]], val.at[j],
                               gsem.at[j])
              for j in range(UNA)]
        sd = []
        for j in range(UNA):
            gd[j].wait()
            sd.append(pltpu.async_copy(val.at[j],
                                       sacc.at[idxb.at[b, 2 * j + 1]],
                                       ssem.at[j], add=True))
        for dsc in sd:
            dsc.wait()
        return carry

    lax.fori_loop(0, _ITERA, sbody, 0)
    plsc.subcore_barrier()

    pltpu.sync_copy(sacc.at[pl.ds(s * NPT, NPT)], stage)

    def abody(k, carry):
        ix = pl.ds(k * 16, 16)
        stage[ix] = disbuf[ix] * (stage[ix] + xbuf[ix])
        return carry

    lax.fori_loop(0, _VPI, abody, 0)

    wlo = wbuf[pl.ds(c * 32, 16)]
    whi = wbuf[pl.ds(c * 32 + 16, 16)]
    blo = bbuf[pl.ds(c * 32, 16)]
    bhi = bbuf[pl.ds(c * 32 + 16, 16)]

    def qbody(ch, carry):
        bb = lax.rem(ch, 2)

        @pl.when(ch >= 2)
        def _drain():
            pltpu.make_async_copy(
                qch.at[pl.ds(bb * NCH, NCH)],
                q_hbm.at[pl.ds(c * N + s * NPT + (ch - 2) * NCH, NCH)],
                qsem.at[bb]).wait()

        def nbody(i, carry2):
            node = jnp.full((16,), ch * NCH + i, jnp.int32)
            a = plsc.load_gather(stage, [node])
            dv = plsc.load_gather(disbuf, [node])
            r = bb * NCH + i
            for (w_, b_, g) in ((wlo, blo, 0), (whi, bhi, 1)):
                z = a * w_ + b_
                z = jnp.maximum(jnp.minimum(z, 15.0), -15.0)
                t = jnp.exp(z + z)
                qch[r, pl.ds(16 * g, 16)] = dv * (t - 1.0) / (t + 1.0)
            return carry2

        lax.fori_loop(0, NCH, nbody, 0)
        pltpu.async_copy(qch.at[pl.ds(bb * NCH, NCH)],
                         q_hbm.at[pl.ds(c * N + s * NPT + ch * NCH, NCH)],
                         qsem.at[bb])
        return carry

    lax.fori_loop(0, NPT // NCH, qbody, 0)
    for bb in range(2):
        pltpu.make_async_copy(
            qch.at[pl.ds(bb * NCH, NCH)],
            q_hbm.at[pl.ds(c * N + s * NPT + (30 + bb) * NCH, NCH)],
            qsem.at[bb]).wait()


@functools.partial(
    pl.kernel,
    out_type=jax.ShapeDtypeStruct((NC * N, 32), jnp.float32),
    mesh=_mesh,
    compiler_params=pltpu.CompilerParams(use_tc_tiling_on_sc=False,
                                         needs_layout_passes=False),
    scratch_types=[
        pltpu.VMEM((2, 2 * UNC, SUB), jnp.int32),
        pltpu.VMEM((UNC, SUB, 32), jnp.float32),
        pltpu.VMEM((NPT // 16, 32), jnp.float32),
        pltpu.VMEM((NPT,), jnp.float32),
        pltpu.VMEM_SHARED((N, 32), jnp.float32),
        pltpu.SemaphoreType.DMA((2,)),
        pltpu.SemaphoreType.DMA((UNC,)),
        pltpu.SemaphoreType.DMA((UNC,)),
    ],
)
def _sc_scat2(sd2_hbm, q_hbm, dis_hbm, out_hbm,
              idxb, rows, stage, disbuf, acc, isem, gsem, ssem):
    c = lax.axis_index("c")
    s = lax.axis_index("s")
    pltpu.sync_copy(dis_hbm.at[pl.ds(c * N + s * NPT, NPT)], disbuf)
    for j in range(16):
        pltpu.sync_copy(
            q_hbm.at[pl.ds(c * N + s * NPT + j * (NPT // 16), NPT // 16)],
            stage)
        pltpu.sync_copy(stage,
                        acc.at[pl.ds(s * NPT + j * (NPT // 16), NPT // 16)])
    plsc.subcore_barrier()
    row0 = c * 2 * (E // SUB) + s * 2 * _ROWS_T
    pltpu.async_copy(sd2_hbm.at[pl.ds(row0, 2 * UNC)], idxb.at[0], isem.at[0])

    def body(k, carry):
        b = lax.rem(k, 2)

        @pl.when(k + 1 < _ITER16)
        def _pref():
            pltpu.async_copy(
                sd2_hbm.at[pl.ds(row0 + (k + 1) * 2 * UNC, 2 * UNC)],
                idxb.at[1 - b], isem.at[1 - b])

        pltpu.make_async_copy(sd2_hbm.at[pl.ds(row0 + k * 2 * UNC, 2 * UNC)],
                              idxb.at[b], isem.at[b]).wait()
        gd = [pltpu.async_copy(q_hbm.at[idxb.at[b, 2 * j]], rows.at[j],
                               gsem.at[j])
              for j in range(UNC)]
        sd = []
        for j in range(UNC):
            gd[j].wait()
            sd.append(pltpu.async_copy(rows.at[j],
                                       acc.at[idxb.at[b, 2 * j + 1]],
                                       ssem.at[j], add=True))
        for dsc in sd:
            dsc.wait()
        return carry

    lax.fori_loop(0, _ITER16, body, 0)
    plsc.subcore_barrier()

    for j in range(16):
        pltpu.sync_copy(acc.at[pl.ds(s * NPT + j * (NPT // 16), NPT // 16)],
                        stage)

        def scbody(i, carry):
            dv = plsc.load_gather(
                disbuf, [jnp.full((16,), j * (NPT // 16) + i, jnp.int32)])
            stage[i, pl.ds(0, 16)] = dv * stage[i, pl.ds(0, 16)]
            stage[i, pl.ds(16, 16)] = dv * stage[i, pl.ds(16, 16)]
            return carry

        lax.fori_loop(0, NPT // 16, scbody, 0)
        pltpu.sync_copy(stage,
                        out_hbm.at[pl.ds(c * N + s * NPT + j * (NPT // 16),
                                         NPT // 16)])


_BP = 1568
_GRID_F = N // (4 * _BP)


def _tc_final_body(scp_ref, batch_ref, w2a_ref, w2b_ref, b2_ref, xr_ref,
                   wfc_ref, bfc_ref, wr1_ref, br1_ref, wr2_ref, br2_ref,
                   fw_ref, out_ref, sums_ref, cnt_ref):
    i = pl.program_id(0)

    @pl.when(i == 0)
    def _init():
        sums_ref[...] = jnp.zeros_like(sums_ref)
        cnt_ref[...] = jnp.zeros_like(cnt_ref)

    h2p = jnp.tanh(
        jnp.dot(scp_ref[0], w2a_ref[...], preferred_element_type=jnp.float32)
        + jnp.dot(scp_ref[1], w2b_ref[...],
                  preferred_element_type=jnp.float32)
        + b2_ref[...])
    bv = batch_ref[0]
    for l in range(4):
        ohl = (bv[l:l + 1, :] ==
               lax.broadcasted_iota(jnp.int32, (G, _BP), 0)
               ).astype(jnp.float32)
        sums_ref[...] += jnp.dot(ohl, h2p[:, 128 * l:128 * (l + 1)],
                                 preferred_element_type=jnp.float32)
        cnt_ref[...] += jnp.sum(ohl, axis=1, keepdims=True)

    @pl.when(i == _GRID_F - 1)
    def _fin():
        pool = sums_ref[...] / jnp.maximum(cnt_ref[...], 1.0)
        h = jnp.dot(pool, wfc_ref[...],
                    preferred_element_type=jnp.float32) + bfc_ref[...]
        r1 = jnp.maximum(
            jnp.dot(xr_ref[...], wr1_ref[...],
                    preferred_element_type=jnp.float32) + br1_ref[...], 0.0)
        r = jnp.dot(r1, wr2_ref[...],
                    preferred_element_type=jnp.float32) + br2_ref[...]
        fw = jax.nn.sigmoid(fw_ref[0, 0])
        fused = fw * h + (1.0 - fw) * r
        m = jnp.max(fused, axis=1, keepdims=True)
        lse = m + jnp.log(jnp.sum(jnp.exp(fused - m), axis=1, keepdims=True))
        out_ref[...] = fused - lse


_tc_final = pl.pallas_call(
    _tc_final_body,
    grid=(_GRID_F,),
    in_specs=[
        pl.BlockSpec((2, _BP, 128), lambda i: (0, i, 0)),
        pl.BlockSpec((1, 4, _BP), lambda i: (i, 0, 0)),
        pl.BlockSpec((128, 512), lambda i: (0, 0)),
        pl.BlockSpec((128, 512), lambda i: (0, 0)),
        pl.BlockSpec((1, 512), lambda i: (0, 0)),
        pl.BlockSpec((G, 784), lambda i: (0, 0)),
        pl.BlockSpec((128, 10), lambda i: (0, 0)),
        pl.BlockSpec((1, 10), lambda i: (0, 0)),
        pl.BlockSpec((784, 128), lambda i: (0, 0)),
        pl.BlockSpec((1, 128), lambda i: (0, 0)),
        pl.BlockSpec((128, 10), lambda i: (0, 0)),
        pl.BlockSpec((1, 10), lambda i: (0, 0)),
        pl.BlockSpec((1, 1), lambda i: (0, 0)),
    ],
    out_specs=pl.BlockSpec((G, 10), lambda i: (0, 0)),
    out_shape=jax.ShapeDtypeStruct((G, 10), jnp.float32),
    scratch_shapes=[
        pltpu.VMEM((G, 128), jnp.float32),
        pltpu.VMEM((G, 1), jnp.float32),
    ],
)


def kernel(x, edge_index, batch, W1, b1, W2, b2, Wfc, bfc, Wr1, br1, Wr2, br2,
           fusion_weight):
    from jax.scipy.linalg import block_diag

    src = edge_index[0]
    dst = edge_index[1]
    dm = dst.reshape(E // SUB, SUB)
    sm = src.reshape(E // SUB, SUB)
    sd1 = jnp.stack([sm, dm], axis=1).reshape(2 * (E // SUB), SUB)
    s2r = jnp.stack([sm, sm + N])
    dmr = jnp.broadcast_to(dm[None], (2,) + dm.shape)
    sd2 = jnp.stack([s2r, dmr], axis=2).reshape(4 * (E // SUB), SUB)

    zeros_n = jnp.zeros((NPT,), jnp.float32)
    ones_sub = jnp.ones((SUB,), jnp.float32)

    q_lin, dis2 = _sc_l1(sd1, dm, x.reshape(N), zeros_n, ones_sub,
                         W1.reshape(64), b1)
    agg2 = _sc_scat2(sd2, q_lin, dis2)

    w2a_t = block_diag(W2[:32], W2[:32], W2[:32], W2[:32])
    w2b_t = block_diag(W2[32:], W2[32:], W2[32:], W2[32:])
    b2rep = jnp.tile(b2, 4).reshape(1, 512)
    batch4 = batch.reshape(_GRID_F, _BP, 4).transpose(0, 2, 1)

    out = _tc_final(agg2.reshape(NC, N // 4, 128), batch4, w2a_t, w2b_t,
                    b2rep, x.reshape(G, 784), Wfc, bfc.reshape(1, 10), Wr1,
                    br1.reshape(1, 128), Wr2, br2.reshape(1, 10),
                    fusion_weight.reshape(1, 1))
    return out


# speedup vs baseline: 69.4012x; 1.0101x over previous
import functools

import jax
import jax.numpy as jnp
from jax import lax
from jax.experimental import pallas as pl
from jax.experimental.pallas import tpu as pltpu
from jax.experimental.pallas import tpu_sc as plsc

N = 50176
E = 802816
G = 64

NC = 2
NS = 16
SUB = 128
UNA = 8
UNC = 4
NPT = N // NS
NCH = NPT // 32

_ROWS_T = E // (NS * SUB)
_ITERA = _ROWS_T // UNA
_ITER16 = _ROWS_T // UNC
_VPI = NPT // 16

_mesh = plsc.VectorSubcoreMesh(core_axis_name="c", subcore_axis_name="s")


@functools.partial(
    pl.kernel,
    out_type=(jax.ShapeDtypeStruct((NC * N, 32), jnp.float32),
              jax.ShapeDtypeStruct((NC * N,), jnp.float32)),
    mesh=_mesh,
    compiler_params=pltpu.CompilerParams(use_tc_tiling_on_sc=False,
                                         needs_layout_passes=False),
    scratch_types=[
        pltpu.VMEM((2, 2 * UNA, SUB), jnp.int32),
        pltpu.VMEM((2, UNA, SUB), jnp.int32),
        pltpu.VMEM((UNA, SUB), jnp.float32),
        pltpu.VMEM((SUB,), jnp.float32),
        pltpu.VMEM((NPT,), jnp.float32),
        pltpu.VMEM((NPT,), jnp.float32),
        pltpu.VMEM((NPT,), jnp.float32),
        pltpu.VMEM((64,), jnp.float32),
        pltpu.VMEM((64,), jnp.float32),
        pltpu.VMEM((2 * NCH, 32), jnp.float32),
        pltpu.VMEM_SHARED((N,), jnp.float32),
        pltpu.VMEM_SHARED((N,), jnp.float32),
        pltpu.SemaphoreType.DMA((2,)),
        pltpu.SemaphoreType.DMA((UNA,)),
        pltpu.SemaphoreType.DMA((UNA,)),
        pltpu.SemaphoreType.DMA((2,)),
    ],
)
def _sc_l1(sd_hbm, dm_hbm, x_hbm, zeros_hbm, ones_hbm, w1_hbm, b1_hbm,
           q_hbm, dis_hbm,
           idxb, didxb, val, ones_v, stage, xbuf, disbuf, wbuf, bbuf, qch,
           dacc, sacc, isem, gsem, ssem, qsem):
    c = lax.axis_index("c")
    s = lax.axis_index("s")
    pltpu.sync_copy(zeros_hbm, stage)
    pltpu.sync_copy(stage, dacc.at[pl.ds(s * NPT, NPT)])
    pltpu.sync_copy(stage, sacc.at[pl.ds(s * NPT, NPT)])
    pltpu.sync_copy(ones_hbm, ones_v)
    pltpu.sync_copy(w1_hbm, wbuf)
    pltpu.sync_copy(b1_hbm, bbuf)
    plsc.subcore_barrier()

    drow0 = s * _ROWS_T
    pltpu.async_copy(dm_hbm.at[pl.ds(drow0, UNA)], didxb.at[0], isem.at[0])

    def dbody(k, carry):
        b = lax.rem(k, 2)

        @pl.when(k + 1 < _ITERA)
        def _pref():
            pltpu.async_copy(dm_hbm.at[pl.ds(drow0 + (k + 1) * UNA, UNA)],
                             didxb.at[1 - b], isem.at[1 - b])

        pltpu.make_async_copy(dm_hbm.at[pl.ds(drow0 + k * UNA, UNA)],
                              didxb.at[b], isem.at[b]).wait()
        descs = [
            pltpu.async_copy(ones_v, dacc.at[didxb.at[b, j]], gsem.at[j],
                             add=True)
            for j in range(UNA)
        ]
        for dsc in descs:
            dsc.wait()
        return carry

    lax.fori_loop(0, _ITERA, dbody, 0)
    plsc.subcore_barrier()

    pltpu.sync_copy(dacc.at[pl.ds(s * NPT, NPT)], stage)
    pltpu.sync_copy(x_hbm.at[pl.ds(s * NPT, NPT)], xbuf)

    def vbody(k, carry):
        ix = pl.ds(k * 16, 16)
        d = stage[ix] + 1.0
        bits = plsc.bitcast(d, jnp.int32)
        y = plsc.bitcast(jnp.int32(0x5F3759DF) - (bits >> 1), jnp.float32)
        y = y * (1.5 - 0.5 * d * y * y)
        y = y * (1.5 - 0.5 * d * y * y)
        y = y * (1.5 - 0.5 * d * y * y)
        disbuf[ix] = y
        xbuf[ix] = y * xbuf[ix]
        return carry

    lax.fori_loop(0, _VPI, vbody, 0)
    pltpu.sync_copy(xbuf, dacc.at[pl.ds(s * NPT, NPT)])
    pltpu.sync_copy(disbuf, dis_hbm.at[pl.ds(c * N + s * NPT, NPT)])
    plsc.subcore_barrier()

    srow0 = s * 2 * _ROWS_T
    pltpu.async_copy(sd_hbm.at[pl.ds(srow0, 2 * UNA)], idxb.at[0], isem.at[0])

    def sbody(k, carry):
        b = lax.rem(k, 2)

        @pl.when(k + 1 < _ITERA)
        def _pref():
            pltpu.async_copy(
                sd_hbm.at[pl.ds(srow0 + (k + 1) * 2 * UNA, 2 * UNA)],
                idxb.at[1 - b], isem.at[1 - b])

        pltpu.make_async_copy(sd_hbm.at[pl.ds(srow0 + k * 2 * UNA, 2 * UNA)],
                              idxb.at[b], isem.at[b]).wait()
        gd = [pltpu.async_copy(dacc.at[idxb.at[b, 2 * j]], val.at[j],
                               gsem.at[j])
              for j in range(UNA)]
        sd = []
        for j in range(UNA):
            gd[j].wait()
            sd.append(pltpu.async_copy(val.at[j],
                                       sacc.at[idxb.at[b, 2 * j + 1]],
                                       ssem.at[j], add=True))
        for dsc in sd:
            dsc.wait()
        return carry

    lax.fori_loop(0, _ITERA, sbody, 0)
    plsc.subcore_barrier()

    pltpu.sync_copy(sacc.at[pl.ds(s * NPT, NPT)], stage)

    def abody(k, carry):
        ix = pl.ds(k * 16, 16)
        stage[ix] = disbuf[ix] * (stage[ix] + xbuf[ix])
        return carry

    lax.fori_loop(0, _VPI, abody, 0)

    wlo = wbuf[pl.ds(c * 32, 16)]
    whi = wbuf[pl.ds(c * 32 + 16, 16)]
    blo = bbuf[pl.ds(c * 32, 16)]
    bhi = bbuf[pl.ds(c * 32 + 16, 16)]

    def qbody(ch, carry):
        bb = lax.rem(ch, 2)

        @pl.when(ch >= 2)
        def _drain():
            pltpu.make_async_copy(
                qch.at[pl.ds(bb * NCH, NCH)],
                q_hbm.at[pl.ds(c * N + s * NPT + (ch - 2) * NCH, NCH)],
                qsem.at[bb]).wait()

        def nbody(i, carry2):
            node = jnp.full((16,), ch * NCH + i, jnp.int32)
            a = plsc.load_gather(stage, [node])
            dv = plsc.load_gather(disbuf, [node])
            r = bb * NCH + i
            for (w_, b_, g) in ((wlo, blo, 0), (whi, bhi, 1)):
                z = a * w_ + b_
                z = jnp.maximum(jnp.minimum(z, 15.0), -15.0)
                t = jnp.exp(z + z)
                qch[r, pl.ds(16 * g, 16)] = dv * (t - 1.0) / (t + 1.0)
            return carry2

        lax.fori_loop(0, NCH, nbody, 0)
        pltpu.async_copy(qch.at[pl.ds(bb * NCH, NCH)],
                         q_hbm.at[pl.ds(c * N + s * NPT + ch * NCH, NCH)],
                         qsem.at[bb])
        return carry

    lax.fori_loop(0, NPT // NCH, qbody, 0)
    for bb in range(2):
        pltpu.make_async_copy(
            qch.at[pl.ds(bb * NCH, NCH)],
            q_hbm.at[pl.ds(c * N + s * NPT + (30 + bb) * NCH, NCH)],
            qsem.at[bb]).wait()


@functools.partial(
    pl.kernel,
    out_type=jax.ShapeDtypeStruct((NC * N, 32), jnp.float32),
    mesh=_mesh,
    compiler_params=pltpu.CompilerParams(use_tc_tiling_on_sc=False,
                                         needs_layout_passes=False),
    scratch_types=[
        pltpu.VMEM((2, 2 * UNC, SUB), jnp.int32),
        pltpu.VMEM((UNC, SUB, 32), jnp.float32),
        pltpu.VMEM((2, NPT // 32, 32), jnp.float32),
        pltpu.VMEM((NPT,), jnp.float32),
        pltpu.VMEM_SHARED((N, 32), jnp.float32),
        pltpu.SemaphoreType.DMA((2,)),
        pltpu.SemaphoreType.DMA((UNC,)),
        pltpu.SemaphoreType.DMA((UNC,)),
    ],
)
def _sc_scat2(sd2_hbm, q_hbm, dis_hbm, out_hbm,
              idxb, rows, stage, disbuf, acc, isem, gsem, ssem):
    c = lax.axis_index("c")
    s = lax.axis_index("s")
    pltpu.sync_copy(dis_hbm.at[pl.ds(c * N + s * NPT, NPT)], disbuf)
    _CK = NPT // 32

    def _qsrc(j):
        return q_hbm.at[pl.ds(c * N + s * NPT + j * _CK, _CK)]

    def _accr(j):
        return acc.at[pl.ds(s * NPT + j * _CK, _CK)]

    pltpu.async_copy(_qsrc(0), stage.at[0], isem.at[0])
    for j in range(32):
        b = j % 2
        if j + 1 < 32:
            if j >= 1:
                pltpu.make_async_copy(stage.at[1 - b], _accr(j - 1),
                                      gsem.at[1 - b]).wait()
            pltpu.async_copy(_qsrc(j + 1), stage.at[1 - b], isem.at[1 - b])
        pltpu.make_async_copy(_qsrc(j), stage.at[b], isem.at[b]).wait()
        pltpu.async_copy(stage.at[b], _accr(j), gsem.at[b])
    for j in (30, 31):
        pltpu.make_async_copy(stage.at[j % 2], _accr(j), gsem.at[j % 2]).wait()
    plsc.subcore_barrier()
    row0 = c * 2 * (E // SUB) + s * 2 * _ROWS_T
    pltpu.async_copy(sd2_hbm.at[pl.ds(row0, 2 * UNC)], idxb.at[0], isem.at[0])

    def body(k, carry):
        b = lax.rem(k, 2)

        @pl.when(k + 1 < _ITER16)
        def _pref():
            pltpu.async_copy(
                sd2_hbm.at[pl.ds(row0 + (k + 1) * 2 * UNC, 2 * UNC)],
                idxb.at[1 - b], isem.at[1 - b])

        pltpu.make_async_copy(sd2_hbm.at[pl.ds(row0 + k * 2 * UNC, 2 * UNC)],
                              idxb.at[b], isem.at[b]).wait()
        gd = [pltpu.async_copy(q_hbm.at[idxb.at[b, 2 * j]], rows.at[j],
                               gsem.at[j])
              for j in range(UNC)]
        sd = []
        for j in range(UNC):
            gd[j].wait()
            sd.append(pltpu.async_copy(rows.at[j],
                                       acc.at[idxb.at[b, 2 * j + 1]],
                                       ssem.at[j], add=True))
        for dsc in sd:
            dsc.wait()
        return carry

    lax.fori_loop(0, _ITER16, body, 0)
    plsc.subcore_barrier()

    def _outr(j):
        return out_hbm.at[pl.ds(c * N + s * NPT + j * _CK, _CK)]

    pltpu.async_copy(_accr(0), stage.at[0], isem.at[0])
    for j in range(32):
        b = j % 2
        if j + 1 < 32:
            if j >= 1:
                pltpu.make_async_copy(stage.at[1 - b], _outr(j - 1),
                                      gsem.at[1 - b]).wait()
            pltpu.async_copy(_accr(j + 1), stage.at[1 - b], isem.at[1 - b])
        pltpu.make_async_copy(_accr(j), stage.at[b], isem.at[b]).wait()

        def scbody(i, carry, _j=j, _b=b):
            dv = plsc.load_gather(
                disbuf, [jnp.full((16,), _j * _CK + i, jnp.int32)])
            stage[_b, i, pl.ds(0, 16)] = dv * stage[_b, i, pl.ds(0, 16)]
            stage[_b, i, pl.ds(16, 16)] = dv * stage[_b, i, pl.ds(16, 16)]
            return carry

        lax.fori_loop(0, _CK, scbody, 0)
        pltpu.async_copy(stage.at[b], _outr(j), gsem.at[b])
    for j in (30, 31):
        pltpu.make_async_copy(stage.at[j % 2], _outr(j), gsem.at[j % 2]).wait()


_BP = 1568
_GRID_F = N // (4 * _BP)


def _tc_final_body(scp_ref, batch_ref, w2a_ref, w2b_ref, b2_ref, xr_ref,
                   wfc_ref, bfc_ref, wr1_ref, br1_ref, wr2_ref, br2_ref,
                   fw_ref, out_ref, sums_ref, cnt_ref):
    i = pl.program_id(0)

    @pl.when(i == 0)
    def _init():
        sums_ref[...] = jnp.zeros_like(sums_ref)
        cnt_ref[...] = jnp.zeros_like(cnt_ref)

    h2p = jnp.tanh(
        jnp.dot(scp_ref[0], w2a_ref[...], preferred_element_type=jnp.float32)
        + jnp.dot(scp_ref[1], w2b_ref[...],
                  preferred_element_type=jnp.float32)
        + b2_ref[...])
    bv = batch_ref[0]
    for l in range(4):
        ohl = (bv[l:l + 1, :] ==
               lax.broadcasted_iota(jnp.int32, (G, _BP), 0)
               ).astype(jnp.float32)
        sums_ref[...] += jnp.dot(ohl, h2p[:, 128 * l:128 * (l + 1)],
                                 preferred_element_type=jnp.float32)
        cnt_ref[...] += jnp.sum(ohl, axis=1, keepdims=True)

    @pl.when(i == _GRID_F - 1)
    def _fin():
        pool = sums_ref[...] / jnp.maximum(cnt_ref[...], 1.0)
        h = jnp.dot(pool, wfc_ref[...],
                    preferred_element_type=jnp.float32) + bfc_ref[...]
        r1 = jnp.maximum(
            jnp.dot(xr_ref[...], wr1_ref[...],
                    preferred_element_type=jnp.float32) + br1_ref[...], 0.0)
        r = jnp.dot(r1, wr2_ref[...],
                    preferred_element_type=jnp.float32) + br2_ref[...]
        fw = jax.nn.sigmoid(fw_ref[0, 0])
        fused = fw * h + (1.0 - fw) * r
        m = jnp.max(fused, axis=1, keepdims=True)
        lse = m + jnp.log(jnp.sum(jnp.exp(fused - m), axis=1, keepdims=True))
        out_ref[...] = fused - lse


_tc_final = pl.pallas_call(
    _tc_final_body,
    grid=(_GRID_F,),
    in_specs=[
        pl.BlockSpec((2, _BP, 128), lambda i: (0, i, 0)),
        pl.BlockSpec((1, 4, _BP), lambda i: (i, 0, 0)),
        pl.BlockSpec((128, 512), lambda i: (0, 0)),
        pl.BlockSpec((128, 512), lambda i: (0, 0)),
        pl.BlockSpec((1, 512), lambda i: (0, 0)),
        pl.BlockSpec((G, 784), lambda i: (0, 0)),
        pl.BlockSpec((128, 10), lambda i: (0, 0)),
        pl.BlockSpec((1, 10), lambda i: (0, 0)),
        pl.BlockSpec((784, 128), lambda i: (0, 0)),
        pl.BlockSpec((1, 128), lambda i: (0, 0)),
        pl.BlockSpec((128, 10), lambda i: (0, 0)),
        pl.BlockSpec((1, 10), lambda i: (0, 0)),
        pl.BlockSpec((1, 1), lambda i: (0, 0)),
    ],
    out_specs=pl.BlockSpec((G, 10), lambda i: (0, 0)),
    out_shape=jax.ShapeDtypeStruct((G, 10), jnp.float32),
    scratch_shapes=[
        pltpu.VMEM((G, 128), jnp.float32),
        pltpu.VMEM((G, 1), jnp.float32),
    ],
)


def kernel(x, edge_index, batch, W1, b1, W2, b2, Wfc, bfc, Wr1, br1, Wr2, br2,
           fusion_weight):
    from jax.scipy.linalg import block_diag

    src = edge_index[0]
    dst = edge_index[1]
    dm = dst.reshape(E // SUB, SUB)
    sm = src.reshape(E // SUB, SUB)
    sd1 = jnp.stack([sm, dm], axis=1).reshape(2 * (E // SUB), SUB)
    s2r = jnp.stack([sm, sm + N])
    dmr = jnp.broadcast_to(dm[None], (2,) + dm.shape)
    sd2 = jnp.stack([s2r, dmr], axis=2).reshape(4 * (E // SUB), SUB)

    zeros_n = jnp.zeros((NPT,), jnp.float32)
    ones_sub = jnp.ones((SUB,), jnp.float32)

    q_lin, dis2 = _sc_l1(sd1, dm, x.reshape(N), zeros_n, ones_sub,
                         W1.reshape(64), b1)
    agg2 = _sc_scat2(sd2, q_lin, dis2)

    w2a_t = block_diag(W2[:32], W2[:32], W2[:32], W2[:32])
    w2b_t = block_diag(W2[32:], W2[32:], W2[32:], W2[32:])
    b2rep = jnp.tile(b2, 4).reshape(1, 512)
    batch4 = batch.reshape(_GRID_F, _BP, 4).transpose(0, 2, 1)

    out = _tc_final(agg2.reshape(NC, N // 4, 128), batch4, w2a_t, w2b_t,
                    b2rep, x.reshape(G, 784), Wfc, bfc.reshape(1, 10), Wr1,
                    br1.reshape(1, 128), Wr2, br2.reshape(1, 10),
                    fusion_weight.reshape(1, 1))
    return out
